# baseline (device time: 240593 ns/iter reference)
import jax
import jax.numpy as jnp
from jax import lax
from jax.experimental import pallas as pl
from jax.experimental.pallas import tpu as pltpu

N_DEV = 32


def kernel(x, w_mat):
    m_per, k = x.shape
    _, n_per = w_mat.shape
    m_glob = N_DEV * m_per

    def body(x_ref, w_ref, out_ref, xg_ref, send_sems, recv_sems):
        my = lax.axis_index("i")
        left = lax.rem(my + N_DEV - 1, N_DEV)
        right = lax.rem(my + 1, N_DEV)

        barrier_sem = pltpu.get_barrier_semaphore()
        for nbr in (left, right):
            pl.semaphore_signal(
                barrier_sem, inc=1,
                device_id=(nbr,), device_id_type=pl.DeviceIdType.MESH,
            )
        pl.semaphore_wait(barrier_sem, 2)

        xg_ref[pl.ds(my * m_per, m_per), :] = x_ref[:, :]

        for h in range(N_DEV - 1):
            o = lax.rem(my + (N_DEV - h), N_DEV)
            sl = pl.ds(o * m_per, m_per)
            rdma = pltpu.make_async_remote_copy(
                src_ref=xg_ref.at[sl, :],
                dst_ref=xg_ref.at[sl, :],
                send_sem=send_sems.at[h],
                recv_sem=recv_sems.at[h],
                device_id=(right,),
                device_id_type=pl.DeviceIdType.MESH,
            )
            rdma.start()
            rdma.wait()

        acc = jnp.dot(xg_ref[:, :], w_ref[:, :],
                      preferred_element_type=jnp.float32)
        out_ref[:, :] = jnp.maximum(acc, 0.0)

    return pl.pallas_call(
        body,
        out_shape=jax.ShapeDtypeStruct((m_glob, n_per), jnp.float32),
        in_specs=[
            pl.BlockSpec(memory_space=pltpu.VMEM),
            pl.BlockSpec(memory_space=pltpu.VMEM),
        ],
        out_specs=pl.BlockSpec(memory_space=pltpu.VMEM),
        scratch_shapes=[
            pltpu.VMEM((m_glob, k), jnp.float32),
            pltpu.SemaphoreType.DMA((N_DEV - 1,)),
            pltpu.SemaphoreType.DMA((N_DEV - 1,)),
        ],
        compiler_params=pltpu.CompilerParams(collective_id=0),
    )(x, w_mat)


# device time: 187861 ns/iter; 1.2807x vs baseline; 1.2807x over previous
import jax
import jax.numpy as jnp
from jax import lax
from jax.experimental import pallas as pl
from jax.experimental.pallas import tpu as pltpu

N_DEV = 32


def kernel(x, w_mat):
    m_per, k = x.shape
    _, n_per = w_mat.shape
    m_glob = N_DEV * m_per
    half = m_per // 2

    def body(x_ref, w_ref, out_ref, xg_ref,
             send_cw, recv_cw, send_ccw, recv_ccw, dummy_sem):
        my = lax.axis_index("i")
        left = lax.rem(my + N_DEV - 1, N_DEV)
        right = lax.rem(my + 1, N_DEV)

        barrier_sem = pltpu.get_barrier_semaphore()
        for nbr in (left, right):
            pl.semaphore_signal(
                barrier_sem, inc=1,
                device_id=(nbr,), device_id_type=pl.DeviceIdType.MESH,
            )
        pl.semaphore_wait(barrier_sem, 2)

        def sl(o, i):
            return pl.ds(o * m_per + i * half, half)

        def chunk_cw(d):
            return lax.rem(my + (N_DEV - d), N_DEV)

        def chunk_ccw(d):
            return lax.rem(my + d, N_DEV)

        sends = []

        def start_send(o, i, send_sem, recv_sem, target):
            s = sl(o, i)
            r = pltpu.make_async_remote_copy(
                src_ref=xg_ref.at[s, :],
                dst_ref=xg_ref.at[s, :],
                send_sem=send_sem,
                recv_sem=recv_sem,
                device_id=(target,),
                device_id_type=pl.DeviceIdType.MESH,
            )
            r.start()
            sends.append(r)

        def wait_recv(o, i, recv_sem):
            s = sl(o, i)
            pltpu.make_async_remote_copy(
                src_ref=xg_ref.at[s, :],
                dst_ref=xg_ref.at[s, :],
                send_sem=dummy_sem.at[0],
                recv_sem=recv_sem,
                device_id=(left,),
                device_id_type=pl.DeviceIdType.MESH,
            ).wait_recv()

        xg_ref[pl.ds(my * m_per, m_per), :] = x_ref[:, :]
        for i in (0, 1):
            start_send(my, i, send_cw.at[0 * 2 + i], recv_cw.at[1 * 2 + i], right)
        for i in (0, 1):
            start_send(my, i, send_ccw.at[0 * 2 + i], recv_ccw.at[1 * 2 + i], left)

        for d in range(1, 16):
            o_cw = chunk_cw(d)
            for i in (0, 1):
                wait_recv(o_cw, i, recv_cw.at[d * 2 + i])
                if d <= 14:
                    start_send(o_cw, i, send_cw.at[d * 2 + i],
                               recv_cw.at[(d + 1) * 2 + i], right)
                elif i == 0:
                    start_send(o_cw, 0, send_cw.at[15 * 2 + 0],
                               recv_cw.at[16 * 2 + 0], right)
            o_ccw = chunk_ccw(d)
            for i in (0, 1):
                wait_recv(o_ccw, i, recv_ccw.at[d * 2 + i])
                if d <= 14:
                    start_send(o_ccw, i, send_ccw.at[d * 2 + i],
                               recv_ccw.at[(d + 1) * 2 + i], left)
                elif i == 1:
                    start_send(o_ccw, 1, send_ccw.at[15 * 2 + 1],
                               recv_ccw.at[16 * 2 + 1], left)

        acc = jnp.dot(xg_ref[:, :], w_ref[:, :],
                      preferred_element_type=jnp.float32)
        out_ref[:, :] = jnp.maximum(acc, 0.0)

        o_anti = lax.rem(my + 16, N_DEV)
        wait_recv(o_anti, 0, recv_cw.at[16 * 2 + 0])
        wait_recv(o_anti, 1, recv_ccw.at[16 * 2 + 1])
        rows = pl.ds(o_anti * m_per, m_per)
        patch = jnp.dot(xg_ref[rows, :], w_ref[:, :],
                        preferred_element_type=jnp.float32)
        out_ref[rows, :] = jnp.maximum(patch, 0.0)

        for r in sends:
            r.wait_send()

    return pl.pallas_call(
        body,
        out_shape=jax.ShapeDtypeStruct((m_glob, n_per), jnp.float32),
        in_specs=[
            pl.BlockSpec(memory_space=pltpu.VMEM),
            pl.BlockSpec(memory_space=pltpu.VMEM),
        ],
        out_specs=pl.BlockSpec(memory_space=pltpu.VMEM),
        scratch_shapes=[
            pltpu.VMEM((m_glob, k), jnp.float32),
            pltpu.SemaphoreType.DMA((32,)),
            pltpu.SemaphoreType.DMA((34,)),
            pltpu.SemaphoreType.DMA((32,)),
            pltpu.SemaphoreType.DMA((34,)),
            pltpu.SemaphoreType.DMA((1,)),
        ],
        compiler_params=pltpu.CompilerParams(collective_id=0),
    )(x, w_mat)
